# baseline (speedup 1.0000x reference)
import functools

import jax
import jax.numpy as jnp
from jax import lax
from jax.experimental import pallas as pl
from jax.experimental.pallas import tpu as pltpu
from jax.experimental.pallas import tpu_sc as plsc

_VOCAB = 100000
_SHARED = 102
_LANG = 26
_D = 128
_B, _L = 4096, 200
_N = _B * _L

_NC, _NS = 2, 16
_NW = _NC * _NS
_PER_W = _N // _NW
_SUB = 128
_C = 256
_NSUB = _C // _SUB
_CHUNKS = _PER_W // _C

_FUSE_ROWS = 2000


def _fuse_body(shared_ref, lang_ref, bias_ref, out_ref):
    out_ref[...] = (
        jnp.concatenate([shared_ref[...], lang_ref[...]], axis=-1) + bias_ref[...]
    )


def _fuse_tables(shared_table, lang_table, bias):
    return pl.pallas_call(
        _fuse_body,
        grid=(_VOCAB // _FUSE_ROWS,),
        in_specs=[
            pl.BlockSpec((_FUSE_ROWS, _SHARED), lambda i: (i, 0)),
            pl.BlockSpec((_FUSE_ROWS, _LANG), lambda i: (i, 0)),
            pl.BlockSpec((1, _D), lambda i: (0, 0)),
        ],
        out_specs=pl.BlockSpec((_FUSE_ROWS, _D), lambda i: (i, 0)),
        out_shape=jax.ShapeDtypeStruct((_VOCAB, _D), jnp.float32),
    )(shared_table, lang_table, bias)


def _gather_body(idx_hbm, table_hbm, out_hbm, idx0, idx1, rows0, rows1, g0, g1):
    wid = lax.axis_index("s") * _NC + lax.axis_index("c")
    idx_row0 = wid * (_PER_W // _SUB)
    out_row0 = wid * _PER_W

    def fire(c, idx_v, rows_v, sem):
        pltpu.sync_copy(idx_hbm.at[pl.ds(idx_row0 + c * _NSUB, _NSUB)], idx_v)
        for j in range(_NSUB):
            pltpu.async_copy(
                table_hbm.at[idx_v.at[j]], rows_v.at[pl.ds(j * _SUB, _SUB)], sem
            )

    def drain(idx_v, rows_v, sem):
        for j in range(_NSUB):
            pltpu.make_async_copy(
                table_hbm.at[idx_v.at[j]], rows_v.at[pl.ds(j * _SUB, _SUB)], sem
            ).wait()

    def write(c, rows_v):
        pltpu.sync_copy(rows_v, out_hbm.at[pl.ds(out_row0 + c * _C, _C)])

    fire(0, idx0, rows0, g0)

    def step(k, carry):
        a = 2 * k
        drain(idx0, rows0, g0)
        fire(a + 1, idx1, rows1, g1)
        write(a, rows0)
        drain(idx1, rows1, g1)
        fire(a + 2, idx0, rows0, g0)
        write(a + 1, rows1)
        return carry

    lax.fori_loop(0, _CHUNKS // 2 - 1, step, 0)

    last = _CHUNKS - 2
    drain(idx0, rows0, g0)
    fire(last + 1, idx1, rows1, g1)
    write(last, rows0)
    drain(idx1, rows1, g1)
    write(last + 1, rows1)


def _gather(idx2d, table):
    mesh = plsc.VectorSubcoreMesh(core_axis_name="c", subcore_axis_name="s")
    run = functools.partial(
        pl.kernel,
        out_type=jax.ShapeDtypeStruct((_N, _D), jnp.float32),
        mesh=mesh,
        scratch_types=[
            pltpu.VMEM((_NSUB, _SUB), jnp.int32),
            pltpu.VMEM((_NSUB, _SUB), jnp.int32),
            pltpu.VMEM((_C, _D), jnp.float32),
            pltpu.VMEM((_C, _D), jnp.float32),
            pltpu.SemaphoreType.DMA,
            pltpu.SemaphoreType.DMA,
        ],
    )(_gather_body)
    return run(idx2d, table)


def kernel(input_ids, shared_table, lang_table, lang_id_table, language_id=0):
    bias = lang_id_table[language_id][None, :]
    table = _fuse_tables(shared_table, lang_table, bias)
    idx2d = input_ids.reshape(_N // _SUB, _SUB).astype(jnp.int32)
    out = _gather(idx2d, table)
    return out.reshape(_B, _L, _D)


# speedup vs baseline: 1.0261x; 1.0261x over previous
import functools

import jax
import jax.numpy as jnp
from jax import lax
from jax.experimental import pallas as pl
from jax.experimental.pallas import tpu as pltpu
from jax.experimental.pallas import tpu_sc as plsc

_VOCAB = 100000
_SHARED = 102
_LANG = 26
_D = 128
_B, _L = 4096, 200
_N = _B * _L

_NC, _NS = 2, 16
_NW = _NC * _NS
_PER_W = _N // _NW
_SUB = 128
_C = 256
_NSUB = _C // _SUB
_CHUNKS = _PER_W // _C

_FUSE_ROWS = 10000


def _fuse_body(shared_ref, lang_ref, bias_ref, out_ref):
    out_ref[...] = (
        jnp.concatenate([shared_ref[...], lang_ref[...]], axis=-1) + bias_ref[...]
    )


def _fuse_tables(shared_table, lang_table, bias):
    return pl.pallas_call(
        _fuse_body,
        grid=(_VOCAB // _FUSE_ROWS,),
        in_specs=[
            pl.BlockSpec((_FUSE_ROWS, _SHARED), lambda i: (i, 0)),
            pl.BlockSpec((_FUSE_ROWS, _LANG), lambda i: (i, 0)),
            pl.BlockSpec((1, _D), lambda i: (0, 0)),
        ],
        out_specs=pl.BlockSpec((_FUSE_ROWS, _D), lambda i: (i, 0)),
        out_shape=jax.ShapeDtypeStruct((_VOCAB, _D), jnp.float32),
    )(shared_table, lang_table, bias)


def _gather_body(idx_hbm, table_hbm, out_hbm, idx0, idx1, rows0, rows1, g0, g1):
    wid = lax.axis_index("s") * _NC + lax.axis_index("c")
    idx_row0 = wid * (_PER_W // _SUB)
    out_row0 = wid * _PER_W

    def fire(c, idx_v, rows_v, sem):
        pltpu.sync_copy(idx_hbm.at[pl.ds(idx_row0 + c * _NSUB, _NSUB)], idx_v)
        for j in range(_NSUB):
            pltpu.async_copy(
                table_hbm.at[idx_v.at[j]], rows_v.at[pl.ds(j * _SUB, _SUB)], sem
            )

    def drain(idx_v, rows_v, sem):
        for j in range(_NSUB):
            pltpu.make_async_copy(
                table_hbm.at[idx_v.at[j]], rows_v.at[pl.ds(j * _SUB, _SUB)], sem
            ).wait()

    def write(c, rows_v):
        pltpu.sync_copy(rows_v, out_hbm.at[pl.ds(out_row0 + c * _C, _C)])

    fire(0, idx0, rows0, g0)

    def step(k, carry):
        a = 2 * k
        drain(idx0, rows0, g0)
        fire(a + 1, idx1, rows1, g1)
        write(a, rows0)
        drain(idx1, rows1, g1)
        fire(a + 2, idx0, rows0, g0)
        write(a + 1, rows1)
        return carry

    lax.fori_loop(0, _CHUNKS // 2 - 1, step, 0)

    last = _CHUNKS - 2
    drain(idx0, rows0, g0)
    fire(last + 1, idx1, rows1, g1)
    write(last, rows0)
    drain(idx1, rows1, g1)
    write(last + 1, rows1)


def _gather(idx2d, table):
    mesh = plsc.VectorSubcoreMesh(core_axis_name="c", subcore_axis_name="s")
    run = functools.partial(
        pl.kernel,
        out_type=jax.ShapeDtypeStruct((_N, _D), jnp.float32),
        mesh=mesh,
        scratch_types=[
            pltpu.VMEM((_NSUB, _SUB), jnp.int32),
            pltpu.VMEM((_NSUB, _SUB), jnp.int32),
            pltpu.VMEM((_C, _D), jnp.float32),
            pltpu.VMEM((_C, _D), jnp.float32),
            pltpu.SemaphoreType.DMA,
            pltpu.SemaphoreType.DMA,
        ],
    )(_gather_body)
    return run(idx2d, table)


def kernel(input_ids, shared_table, lang_table, lang_id_table, language_id=0):
    bias = lang_id_table[language_id][None, :]
    table = _fuse_tables(shared_table, lang_table, bias)
    idx2d = input_ids.reshape(_N // _SUB, _SUB).astype(jnp.int32)
    out = _gather(idx2d, table)
    return out.reshape(_B, _L, _D)
